# Initial kernel scaffold; baseline (speedup 1.0000x reference)
import jax
import jax.numpy as jnp
from jax.experimental import pallas as pl


def kernel(x, edge_index, W_embed, b_embed, W1, b1, W2, b2, W_cls, b_cls):
    raise NotImplementedError("write your pallas kernel here")


# speedup vs baseline: 5.6094x; 5.6094x over previous
import functools

import jax
import jax.numpy as jnp
from jax import lax
from jax.experimental import pallas as pl
from jax.experimental.pallas import tpu as pltpu
from jax.experimental.pallas import tpu_sc as plsc

N_NODES = 10000
N_EDGES = 320000
D = 128
NC = 2
NS = 16
NW = NC * NS
EDGES_PER_W = N_EDGES // NW
CHUNK = 128
NCHUNK = EDGES_PER_W // CHUNK
TAIL = EDGES_PER_W - NCHUNK * CHUNK
SENT = 2**31 - 1

RB_N = 800
NB = 13
LAST_B = N_NODES - (NB - 1) * RB_N
DEG_SPLIT = 8000


def _fill_rows(ref, n_rows, width, value):
  vec = jnp.full((16,), value, jnp.float32)

  def body(j, _):
    for k in range(width // 16):
      ref[j, pl.ds(k * 16, 16)] = vec
    return 0

  lax.fori_loop(0, n_rows, body, 0)


def _zero_slab(zbuf, acc, aoff, rows, zrows):
  off = 0
  while rows > 0:
    n = min(zrows, rows)
    pltpu.sync_copy(zbuf.at[pl.ds(0, n)], acc.at[pl.ds(aoff + off, n)])
    off += n
    rows -= n


def _route(didx, didxb, size, bounds):
  for j in range(size // 16):
    v = didx[pl.ds(j * 16, 16)]
    for b, (lo, hi) in enumerate(bounds):
      m = (v >= lo) & (v < hi)
      didxb[b, pl.ds(j * 16, 16)] = jnp.where(m, v - lo, SENT)


def _scatter_rows(src, accs, didxb, size):
  for b in range(len(accs)):
    pltpu.sync_copy(
        src.at[pl.ds(0, size)],
        accs[b].at[plsc.Indices(didxb.at[b, pl.ds(0, size)],
                                ignored_value=SENT)],
        add=True,
    )


_DEG_SLABS = (
    [(k, 0, k * 1000, k * 1000, 1000) for k in range(8)]
    + [(8, 1, 0, 8000, 1000), (9, 1, 1000, 9000, 1000)]
)


def _deg_body(dst_hbm, out_hbm, didx, didxb, ones_v, zbuf, acc0, acc1):
  c = lax.axis_index("c")
  s = lax.axis_index("s")
  wid = c * NS + s
  accs = (acc0, acc1)
  bounds = ((0, DEG_SPLIT), (DEG_SPLIT, N_NODES))

  _fill_rows(zbuf, 128, 16, 0.0)
  _fill_rows(ones_v, CHUNK, 16, 1.0)
  for (k, a, aoff, _, rows) in _DEG_SLABS:
    @pl.when(s == k)
    def _(a=a, aoff=aoff, rows=rows):
      _zero_slab(zbuf, accs[a], aoff, rows, 128)
  plsc.subcore_barrier()

  base = wid * EDGES_PER_W

  def chunk(off, size):
    pltpu.sync_copy(dst_hbm.at[pl.ds(off, size)], didx.at[pl.ds(0, size)])
    _route(didx, didxb, size, bounds)
    _scatter_rows(ones_v, accs, didxb, size)

  def body(i, _):
    chunk(base + i * CHUNK, CHUNK)
    return 0

  lax.fori_loop(0, NCHUNK, body, 0)
  chunk(base + NCHUNK * CHUNK, TAIL)

  plsc.subcore_barrier()
  for (k, a, aoff, ooff, rows) in _DEG_SLABS:
    @pl.when(s == k)
    def _(a=a, aoff=aoff, ooff=ooff, rows=rows):
      pltpu.sync_copy(accs[a].at[pl.ds(aoff, rows)],
                      out_hbm.at[c, pl.ds(ooff, rows)])


_deg_sc = functools.partial(
    pl.kernel,
    out_type=jax.ShapeDtypeStruct((NC, N_NODES, 16), jnp.float32),
    mesh=plsc.VectorSubcoreMesh(core_axis_name="c", subcore_axis_name="s"),
    scratch_types=[
        pltpu.VMEM((CHUNK,), jnp.int32),
        pltpu.VMEM((2, CHUNK), jnp.int32),
        pltpu.VMEM((CHUNK, 16), jnp.float32),
        pltpu.VMEM((128, 16), jnp.float32),
        pltpu.VMEM_SHARED((DEG_SPLIT, 16), jnp.float32),
        pltpu.VMEM_SHARED((N_NODES - DEG_SPLIT, 16), jnp.float32),
    ],
)(_deg_body)


def _agg_body(y_hbm, src_hbm, dst_hbm, out_hbm, sidx, didx, didxb, rows_v,
              zbuf, gsem, *accs):
  c = lax.axis_index("c")
  s = lax.axis_index("s")
  wid = c * NS + s
  bounds = tuple((b * RB_N, (b + 1) * RB_N) for b in range(NB))

  _fill_rows(zbuf, 128, D, 0.0)
  for k in range(NB):
    @pl.when(s == k)
    def _(k=k):
      _zero_slab(zbuf, accs[k], 0, RB_N if k < NB - 1 else LAST_B, 128)
  plsc.subcore_barrier()

  base = wid * EDGES_PER_W

  def chunk(off, size):
    pltpu.sync_copy(src_hbm.at[pl.ds(off, size)], sidx.at[pl.ds(0, size)])
    pltpu.sync_copy(dst_hbm.at[pl.ds(off, size)], didx.at[pl.ds(0, size)])
    _route(didx, didxb, size, bounds)
    pltpu.async_copy(
        y_hbm.at[sidx.at[pl.ds(0, size)]], rows_v.at[pl.ds(0, size)], gsem
    ).wait()
    _scatter_rows(rows_v, accs, didxb, size)

  def body(i, _):
    chunk(base + i * CHUNK, CHUNK)
    return 0

  lax.fori_loop(0, NCHUNK, body, 0)
  chunk(base + NCHUNK * CHUNK, TAIL)

  plsc.subcore_barrier()
  for k in range(NB):
    @pl.when(s == k)
    def _(k=k):
      sz = RB_N if k < NB - 1 else LAST_B
      pltpu.sync_copy(accs[k], out_hbm.at[c, pl.ds(k * RB_N, sz)])


_agg_sc = functools.partial(
    pl.kernel,
    out_type=jax.ShapeDtypeStruct((NC, N_NODES, D), jnp.float32),
    mesh=plsc.VectorSubcoreMesh(core_axis_name="c", subcore_axis_name="s"),
    scratch_types=[
        pltpu.VMEM((CHUNK,), jnp.int32),
        pltpu.VMEM((CHUNK,), jnp.int32),
        pltpu.VMEM((NB, CHUNK), jnp.int32),
        pltpu.VMEM((CHUNK, D), jnp.float32),
        pltpu.VMEM((128, D), jnp.float32),
        pltpu.SemaphoreType.DMA,
    ] + [pltpu.VMEM_SHARED((RB_N, D), jnp.float32) for _ in range(NB - 1)]
    + [pltpu.VMEM_SHARED((LAST_B, D), jnp.float32)],
)(_agg_body)


RB = 1000
GRID = N_NODES // RB


def _tc1_body(x_ref, we_ref, be_ref, w1_ref, dp_ref, y1_ref, dv_ref):
  h0 = jnp.dot(x_ref[...], we_ref[...], preferred_element_type=jnp.float32)
  h0 = jnp.maximum(h0 + be_ref[...], 0.0)
  dp = dp_ref[...]
  deg = 1.0 + dp[0, :, 0:1] + dp[1, :, 0:1]
  dv = jnp.broadcast_to(lax.rsqrt(deg), (RB, D))
  xw = jnp.dot(h0, w1_ref[...], preferred_element_type=jnp.float32)
  y1_ref[...] = xw * dv
  dv_ref[...] = dv


def _tc2_body(p_ref, y_ref, dv_ref, b_ref, w_ref, out_ref):
  dv = dv_ref[...]
  h = dv * (p_ref[0] + p_ref[1] + y_ref[...]) + b_ref[...]
  out_ref[...] = jnp.dot(h, w_ref[...], preferred_element_type=jnp.float32) * dv


def _tc3_body(p_ref, y_ref, dv_ref, b_ref, wc_ref, bc_ref, out_ref):
  dv = dv_ref[...]
  h = dv * (p_ref[0] + p_ref[1] + y_ref[...]) + b_ref[...]
  out_ref[...] = (
      jnp.dot(h, wc_ref[...], preferred_element_type=jnp.float32) + bc_ref[...]
      )


def _row_spec():
  return pl.BlockSpec((RB, D), lambda i: (i, 0))


def _full_spec(shape):
  return pl.BlockSpec(shape, lambda i: tuple(0 for _ in shape))


def kernel(x, edge_index, W_embed, b_embed, W1, b1, W2, b2, W_cls, b_cls):
  src = edge_index[0]
  dst = edge_index[1]
  be = b_embed.reshape(1, D)
  b1r = b1.reshape(1, D)
  b2r = b2.reshape(1, D)
  ncls = W_cls.shape[1]
  bcr = b_cls.reshape(1, ncls)

  degp = _deg_sc(dst)

  part_spec = pl.BlockSpec((NC, RB, D), lambda i: (0, i, 0))

  y1, dv = pl.pallas_call(
      _tc1_body,
      grid=(GRID,),
      in_specs=[
          _row_spec(),
          _full_spec((D, D)),
          _full_spec((1, D)),
          _full_spec((D, D)),
          pl.BlockSpec((NC, RB, 16), lambda i: (0, i, 0)),
      ],
      out_specs=[_row_spec(), _row_spec()],
      out_shape=[
          jax.ShapeDtypeStruct((N_NODES, D), jnp.float32),
          jax.ShapeDtypeStruct((N_NODES, D), jnp.float32),
      ],
  )(x, W_embed, be, W1, degp)

  p = _agg_sc(y1, src, dst)

  y2 = pl.pallas_call(
      _tc2_body,
      grid=(GRID,),
      in_specs=[
          part_spec, _row_spec(), _row_spec(),
          _full_spec((1, D)), _full_spec((D, D)),
      ],
      out_specs=_row_spec(),
      out_shape=jax.ShapeDtypeStruct((N_NODES, D), jnp.float32),
  )(p, y1, dv, b1r, W2)

  q = _agg_sc(y2, src, dst)

  logits = pl.pallas_call(
      _tc3_body,
      grid=(GRID,),
      in_specs=[
          part_spec, _row_spec(), _row_spec(),
          _full_spec((1, D)), _full_spec((D, ncls)), _full_spec((1, ncls)),
      ],
      out_specs=pl.BlockSpec((RB, ncls), lambda i: (i, 0)),
      out_shape=jax.ShapeDtypeStruct((N_NODES, ncls), jnp.float32),
  )(q, y2, dv, b2r, W_cls, bcr)

  return logits


# speedup vs baseline: 6.2419x; 1.1128x over previous
import functools

import jax
import jax.numpy as jnp
from jax import lax
from jax.experimental import pallas as pl
from jax.experimental.pallas import tpu as pltpu
from jax.experimental.pallas import tpu_sc as plsc

N_NODES = 10000
N_EDGES = 320000
D = 128
NC = 2
NS = 16
NW = NC * NS
EDGES_PER_W = N_EDGES // NW
CHUNK = 128
NCHUNK = EDGES_PER_W // CHUNK
TAIL = EDGES_PER_W - NCHUNK * CHUNK
SENT = 2**31 - 1

RB_N = 800
NB = 13
LAST_B = N_NODES - (NB - 1) * RB_N
DEG_SPLIT = 8000


def _fill_rows(ref, n_rows, width, value):
  vec = jnp.full((16,), value, jnp.float32)

  def body(j, _):
    for k in range(width // 16):
      ref[j, pl.ds(k * 16, 16)] = vec
    return 0

  lax.fori_loop(0, n_rows, body, 0)


def _zero_slab(zbuf, acc, aoff, rows, zrows):
  off = 0
  while rows > 0:
    n = min(zrows, rows)
    pltpu.sync_copy(zbuf.at[pl.ds(0, n)], acc.at[pl.ds(aoff + off, n)])
    off += n
    rows -= n


def _route(didx, didxb, size, bounds):
  for j in range(size // 16):
    v = didx[pl.ds(j * 16, 16)]
    for b, (lo, hi) in enumerate(bounds):
      m = (v >= lo) & (v < hi)
      didxb[b, pl.ds(j * 16, 16)] = jnp.where(m, v - lo, SENT)


def _scatter_rows(src, accs, didxb, size, sem):
  copies = [
      pltpu.async_copy(
          src.at[pl.ds(0, size)],
          accs[b].at[plsc.Indices(didxb.at[b, pl.ds(0, size)],
                                  ignored_value=SENT)],
          sem,
          add=True,
      )
      for b in range(len(accs))
  ]
  for cp in copies:
    cp.wait()


_DEG_SLABS = (
    [(k, 0, k * 1000, k * 1000, 1000) for k in range(8)]
    + [(8, 1, 0, 8000, 1000), (9, 1, 1000, 9000, 1000)]
)


def _deg_body(dst_hbm, out_hbm, didx, didxb, ones_v, zbuf, ssem, acc0, acc1):
  c = lax.axis_index("c")
  s = lax.axis_index("s")
  wid = c * NS + s
  accs = (acc0, acc1)
  bounds = ((0, DEG_SPLIT), (DEG_SPLIT, N_NODES))

  _fill_rows(zbuf, 128, 16, 0.0)
  _fill_rows(ones_v, CHUNK, 16, 1.0)
  for (k, a, aoff, _, rows) in _DEG_SLABS:
    @pl.when(s == k)
    def _(a=a, aoff=aoff, rows=rows):
      _zero_slab(zbuf, accs[a], aoff, rows, 128)
  plsc.subcore_barrier()

  base = wid * EDGES_PER_W

  def chunk(off, size):
    pltpu.sync_copy(dst_hbm.at[pl.ds(off, size)], didx.at[pl.ds(0, size)])
    _route(didx, didxb, size, bounds)
    _scatter_rows(ones_v, accs, didxb, size, ssem)

  def body(i, _):
    chunk(base + i * CHUNK, CHUNK)
    return 0

  lax.fori_loop(0, NCHUNK, body, 0)
  chunk(base + NCHUNK * CHUNK, TAIL)

  plsc.subcore_barrier()
  for (k, a, aoff, ooff, rows) in _DEG_SLABS:
    @pl.when(s == k)
    def _(a=a, aoff=aoff, ooff=ooff, rows=rows):
      pltpu.sync_copy(accs[a].at[pl.ds(aoff, rows)],
                      out_hbm.at[c, pl.ds(ooff, rows)])


_deg_sc = functools.partial(
    pl.kernel,
    out_type=jax.ShapeDtypeStruct((NC, N_NODES, 16), jnp.float32),
    mesh=plsc.VectorSubcoreMesh(core_axis_name="c", subcore_axis_name="s"),
    scratch_types=[
        pltpu.VMEM((CHUNK,), jnp.int32),
        pltpu.VMEM((2, CHUNK), jnp.int32),
        pltpu.VMEM((CHUNK, 16), jnp.float32),
        pltpu.VMEM((128, 16), jnp.float32),
        pltpu.SemaphoreType.DMA,
        pltpu.VMEM_SHARED((DEG_SPLIT, 16), jnp.float32),
        pltpu.VMEM_SHARED((N_NODES - DEG_SPLIT, 16), jnp.float32),
    ],
)(_deg_body)


def _agg_body(y_hbm, src_hbm, dst_hbm, out_hbm, sidx, didx, didxb, rows_v,
              zbuf, gsem, ssem, *accs):
  c = lax.axis_index("c")
  s = lax.axis_index("s")
  wid = c * NS + s
  bounds = tuple((b * RB_N, (b + 1) * RB_N) for b in range(NB))

  _fill_rows(zbuf, 128, D, 0.0)
  for k in range(NB):
    @pl.when(s == k)
    def _(k=k):
      _zero_slab(zbuf, accs[k], 0, RB_N if k < NB - 1 else LAST_B, 128)
  plsc.subcore_barrier()

  base = wid * EDGES_PER_W

  def chunk(off, size):
    pltpu.sync_copy(src_hbm.at[pl.ds(off, size)], sidx.at[pl.ds(0, size)])
    gather = pltpu.async_copy(
        y_hbm.at[sidx.at[pl.ds(0, size)]], rows_v.at[pl.ds(0, size)], gsem)
    pltpu.sync_copy(dst_hbm.at[pl.ds(off, size)], didx.at[pl.ds(0, size)])
    _route(didx, didxb, size, bounds)
    gather.wait()
    _scatter_rows(rows_v, accs, didxb, size, ssem)

  def body(i, _):
    chunk(base + i * CHUNK, CHUNK)
    return 0

  lax.fori_loop(0, NCHUNK, body, 0)
  chunk(base + NCHUNK * CHUNK, TAIL)

  plsc.subcore_barrier()
  for k in range(NB):
    @pl.when(s == k)
    def _(k=k):
      sz = RB_N if k < NB - 1 else LAST_B
      pltpu.sync_copy(accs[k], out_hbm.at[c, pl.ds(k * RB_N, sz)])


_agg_sc = functools.partial(
    pl.kernel,
    out_type=jax.ShapeDtypeStruct((NC, N_NODES, D), jnp.float32),
    mesh=plsc.VectorSubcoreMesh(core_axis_name="c", subcore_axis_name="s"),
    scratch_types=[
        pltpu.VMEM((CHUNK,), jnp.int32),
        pltpu.VMEM((CHUNK,), jnp.int32),
        pltpu.VMEM((NB, CHUNK), jnp.int32),
        pltpu.VMEM((CHUNK, D), jnp.float32),
        pltpu.VMEM((128, D), jnp.float32),
        pltpu.SemaphoreType.DMA,
        pltpu.SemaphoreType.DMA,
    ] + [pltpu.VMEM_SHARED((RB_N, D), jnp.float32) for _ in range(NB - 1)]
    + [pltpu.VMEM_SHARED((LAST_B, D), jnp.float32)],
)(_agg_body)


RB = 1000
GRID = N_NODES // RB


def _tc1_body(x_ref, we_ref, be_ref, w1_ref, dp_ref, y1_ref, dv_ref):
  h0 = jnp.dot(x_ref[...], we_ref[...], preferred_element_type=jnp.float32)
  h0 = jnp.maximum(h0 + be_ref[...], 0.0)
  dp = dp_ref[...]
  deg = 1.0 + dp[0, :, 0:1] + dp[1, :, 0:1]
  dv = jnp.broadcast_to(lax.rsqrt(deg), (RB, D))
  xw = jnp.dot(h0, w1_ref[...], preferred_element_type=jnp.float32)
  y1_ref[...] = xw * dv
  dv_ref[...] = dv


def _tc2_body(p_ref, y_ref, dv_ref, b_ref, w_ref, out_ref):
  dv = dv_ref[...]
  h = dv * (p_ref[0] + p_ref[1] + y_ref[...]) + b_ref[...]
  out_ref[...] = jnp.dot(h, w_ref[...], preferred_element_type=jnp.float32) * dv


def _tc3_body(p_ref, y_ref, dv_ref, b_ref, wc_ref, bc_ref, out_ref):
  dv = dv_ref[...]
  h = dv * (p_ref[0] + p_ref[1] + y_ref[...]) + b_ref[...]
  out_ref[...] = (
      jnp.dot(h, wc_ref[...], preferred_element_type=jnp.float32) + bc_ref[...]
      )


def _row_spec():
  return pl.BlockSpec((RB, D), lambda i: (i, 0))


def _full_spec(shape):
  return pl.BlockSpec(shape, lambda i: tuple(0 for _ in shape))


def kernel(x, edge_index, W_embed, b_embed, W1, b1, W2, b2, W_cls, b_cls):
  src = edge_index[0]
  dst = edge_index[1]
  be = b_embed.reshape(1, D)
  b1r = b1.reshape(1, D)
  b2r = b2.reshape(1, D)
  ncls = W_cls.shape[1]
  bcr = b_cls.reshape(1, ncls)

  degp = _deg_sc(dst)

  part_spec = pl.BlockSpec((NC, RB, D), lambda i: (0, i, 0))

  y1, dv = pl.pallas_call(
      _tc1_body,
      grid=(GRID,),
      in_specs=[
          _row_spec(),
          _full_spec((D, D)),
          _full_spec((1, D)),
          _full_spec((D, D)),
          pl.BlockSpec((NC, RB, 16), lambda i: (0, i, 0)),
      ],
      out_specs=[_row_spec(), _row_spec()],
      out_shape=[
          jax.ShapeDtypeStruct((N_NODES, D), jnp.float32),
          jax.ShapeDtypeStruct((N_NODES, D), jnp.float32),
      ],
  )(x, W_embed, be, W1, degp)

  p = _agg_sc(y1, src, dst)

  y2 = pl.pallas_call(
      _tc2_body,
      grid=(GRID,),
      in_specs=[
          part_spec, _row_spec(), _row_spec(),
          _full_spec((1, D)), _full_spec((D, D)),
      ],
      out_specs=_row_spec(),
      out_shape=jax.ShapeDtypeStruct((N_NODES, D), jnp.float32),
  )(p, y1, dv, b1r, W2)

  q = _agg_sc(y2, src, dst)

  logits = pl.pallas_call(
      _tc3_body,
      grid=(GRID,),
      in_specs=[
          part_spec, _row_spec(), _row_spec(),
          _full_spec((1, D)), _full_spec((D, ncls)), _full_spec((1, ncls)),
      ],
      out_specs=pl.BlockSpec((RB, ncls), lambda i: (i, 0)),
      out_shape=jax.ShapeDtypeStruct((N_NODES, ncls), jnp.float32),
  )(q, y2, dv, b2r, W_cls, bcr)

  return logits


# speedup vs baseline: 6.7308x; 1.0783x over previous
import functools

import jax
import jax.numpy as jnp
from jax import lax
from jax.experimental import pallas as pl
from jax.experimental.pallas import tpu as pltpu
from jax.experimental.pallas import tpu_sc as plsc

N_NODES = 10000
N_EDGES = 320000
D = 128
NC = 2
NS = 16
NW = NC * NS
EDGES_PER_W = N_EDGES // NW
CHUNK = 128
NCHUNK = EDGES_PER_W // CHUNK
TAIL = EDGES_PER_W - NCHUNK * CHUNK
SENT = 2**31 - 1

RB_N = 800
NB = 13
LAST_B = N_NODES - (NB - 1) * RB_N
DEG_SPLIT = 8000


def _fill_rows(ref, n_rows, width, value):
  vec = jnp.full((16,), value, jnp.float32)

  def body(j, _):
    for k in range(width // 16):
      ref[j, pl.ds(k * 16, 16)] = vec
    return 0

  lax.fori_loop(0, n_rows, body, 0)


def _zero_slab(zbuf, acc, aoff, rows, zrows):
  off = 0
  while rows > 0:
    n = min(zrows, rows)
    pltpu.sync_copy(zbuf.at[pl.ds(0, n)], acc.at[pl.ds(aoff + off, n)])
    off += n
    rows -= n


def _route(didx, didxb, size, bounds):
  for j in range(size // 16):
    v = didx[pl.ds(j * 16, 16)]
    for b, (lo, hi) in enumerate(bounds):
      m = (v >= lo) & (v < hi)
      didxb[b, pl.ds(j * 16, 16)] = jnp.where(m, v - lo, SENT)


def _scatter_rows(src, accs, didxb, size, sem):
  copies = [
      pltpu.async_copy(
          src.at[pl.ds(0, size)],
          accs[b].at[plsc.Indices(didxb.at[b, pl.ds(0, size)],
                                  ignored_value=SENT)],
          sem,
          add=True,
      )
      for b in range(len(accs))
  ]
  for cp in copies:
    cp.wait()


_DEG_SLABS = (
    [(k, 0, k * 1000, k * 1000, 1000) for k in range(8)]
    + [(8, 1, 0, 8000, 1000), (9, 1, 1000, 9000, 1000)]
)


def _deg_body(dst_hbm, out_hbm, didx, didxb, ones_v, zbuf, ssem, acc0, acc1):
  c = lax.axis_index("c")
  s = lax.axis_index("s")
  wid = c * NS + s
  accs = (acc0, acc1)
  bounds = ((0, DEG_SPLIT), (DEG_SPLIT, N_NODES))

  _fill_rows(zbuf, 128, 16, 0.0)
  _fill_rows(ones_v, CHUNK, 16, 1.0)
  for (k, a, aoff, _, rows) in _DEG_SLABS:
    @pl.when(s == k)
    def _(a=a, aoff=aoff, rows=rows):
      _zero_slab(zbuf, accs[a], aoff, rows, 128)
  plsc.subcore_barrier()

  base = wid * EDGES_PER_W

  def chunk(off, size):
    pltpu.sync_copy(dst_hbm.at[pl.ds(off, size)], didx.at[pl.ds(0, size)])
    _route(didx, didxb, size, bounds)
    _scatter_rows(ones_v, accs, didxb, size, ssem)

  def body(i, _):
    chunk(base + i * CHUNK, CHUNK)
    return 0

  lax.fori_loop(0, NCHUNK, body, 0)
  chunk(base + NCHUNK * CHUNK, TAIL)

  plsc.subcore_barrier()
  for (k, a, aoff, ooff, rows) in _DEG_SLABS:
    @pl.when(s == k)
    def _(a=a, aoff=aoff, ooff=ooff, rows=rows):
      pltpu.sync_copy(accs[a].at[pl.ds(aoff, rows)],
                      out_hbm.at[c, pl.ds(ooff, rows)])


_deg_sc = functools.partial(
    pl.kernel,
    out_type=jax.ShapeDtypeStruct((NC, N_NODES, 16), jnp.float32),
    mesh=plsc.VectorSubcoreMesh(core_axis_name="c", subcore_axis_name="s"),
    scratch_types=[
        pltpu.VMEM((CHUNK,), jnp.int32),
        pltpu.VMEM((2, CHUNK), jnp.int32),
        pltpu.VMEM((CHUNK, 16), jnp.float32),
        pltpu.VMEM((128, 16), jnp.float32),
        pltpu.SemaphoreType.DMA,
        pltpu.VMEM_SHARED((DEG_SPLIT, 16), jnp.float32),
        pltpu.VMEM_SHARED((N_NODES - DEG_SPLIT, 16), jnp.float32),
    ],
)(_deg_body)


def _agg_body(y_hbm, src_hbm, dst_hbm, out_hbm, sidx0, sidx1, didx, didxb0,
              didxb1, rows0, rows1, zbuf, gsem0, gsem1, ssem, *accs):
  c = lax.axis_index("c")
  s = lax.axis_index("s")
  wid = c * NS + s
  bounds = tuple((b * RB_N, (b + 1) * RB_N) for b in range(NB))
  bufs = ((sidx0, didxb0, rows0, gsem0), (sidx1, didxb1, rows1, gsem1))

  _fill_rows(zbuf, 64, D, 0.0)
  for k in range(NB):
    @pl.when(s == k)
    def _(k=k):
      _zero_slab(zbuf, accs[k], 0, RB_N if k < NB - 1 else LAST_B, 64)
  plsc.subcore_barrier()

  base = wid * EDGES_PER_W

  def prefetch(off, buf):
    sidx, didxb, rows, gsem = buf
    pltpu.sync_copy(src_hbm.at[pl.ds(off, CHUNK)], sidx)
    gather = pltpu.async_copy(y_hbm.at[sidx], rows, gsem)
    pltpu.sync_copy(dst_hbm.at[pl.ds(off, CHUNK)], didx)
    _route(didx, didxb, CHUNK, bounds)
    return gather

  def finish(buf):
    sidx, didxb, rows, gsem = buf
    pltpu.make_async_copy(y_hbm.at[sidx], rows, gsem).wait()
    _scatter_rows(rows, accs, didxb, CHUNK, ssem)

  prefetch(base, bufs[0])

  def body(i, _):
    for par in range(2):
      @pl.when(i % 2 == par)
      def _(par=par):
        @pl.when(i < NCHUNK - 1)
        def _(par=par):
          prefetch(base + (i + 1) * CHUNK, bufs[1 - par])
        finish(bufs[par])
    return 0

  lax.fori_loop(0, NCHUNK, body, 0)

  toff = base + NCHUNK * CHUNK
  pltpu.sync_copy(src_hbm.at[pl.ds(toff, TAIL)], sidx0.at[pl.ds(0, TAIL)])
  gather = pltpu.async_copy(
      y_hbm.at[sidx0.at[pl.ds(0, TAIL)]], rows0.at[pl.ds(0, TAIL)], gsem0)
  pltpu.sync_copy(dst_hbm.at[pl.ds(toff, TAIL)], didx.at[pl.ds(0, TAIL)])
  _route(didx, didxb0, TAIL, bounds)
  gather.wait()
  _scatter_rows(rows0, accs, didxb0, TAIL, ssem)

  plsc.subcore_barrier()
  for k in range(NB):
    @pl.when(s == k)
    def _(k=k):
      sz = RB_N if k < NB - 1 else LAST_B
      pltpu.sync_copy(accs[k], out_hbm.at[c, pl.ds(k * RB_N, sz)])


_agg_sc = functools.partial(
    pl.kernel,
    out_type=jax.ShapeDtypeStruct((NC, N_NODES, D), jnp.float32),
    mesh=plsc.VectorSubcoreMesh(core_axis_name="c", subcore_axis_name="s"),
    scratch_types=[
        pltpu.VMEM((CHUNK,), jnp.int32),
        pltpu.VMEM((CHUNK,), jnp.int32),
        pltpu.VMEM((CHUNK,), jnp.int32),
        pltpu.VMEM((NB, CHUNK), jnp.int32),
        pltpu.VMEM((NB, CHUNK), jnp.int32),
        pltpu.VMEM((CHUNK, D), jnp.float32),
        pltpu.VMEM((CHUNK, D), jnp.float32),
        pltpu.VMEM((64, D), jnp.float32),
        pltpu.SemaphoreType.DMA,
        pltpu.SemaphoreType.DMA,
        pltpu.SemaphoreType.DMA,
    ] + [pltpu.VMEM_SHARED((RB_N, D), jnp.float32) for _ in range(NB - 1)]
    + [pltpu.VMEM_SHARED((LAST_B, D), jnp.float32)],
)(_agg_body)


RB = 1000
GRID = N_NODES // RB


def _tc1_body(x_ref, we_ref, be_ref, w1_ref, dp_ref, y1_ref, dv_ref):
  h0 = jnp.dot(x_ref[...], we_ref[...], preferred_element_type=jnp.float32)
  h0 = jnp.maximum(h0 + be_ref[...], 0.0)
  dp = dp_ref[...]
  deg = 1.0 + dp[0, :, 0:1] + dp[1, :, 0:1]
  dv = jnp.broadcast_to(lax.rsqrt(deg), (RB, D))
  xw = jnp.dot(h0, w1_ref[...], preferred_element_type=jnp.float32)
  y1_ref[...] = xw * dv
  dv_ref[...] = dv


def _tc2_body(p_ref, y_ref, dv_ref, b_ref, w_ref, out_ref):
  dv = dv_ref[...]
  h = dv * (p_ref[0] + p_ref[1] + y_ref[...]) + b_ref[...]
  out_ref[...] = jnp.dot(h, w_ref[...], preferred_element_type=jnp.float32) * dv


def _tc3_body(p_ref, y_ref, dv_ref, b_ref, wc_ref, bc_ref, out_ref):
  dv = dv_ref[...]
  h = dv * (p_ref[0] + p_ref[1] + y_ref[...]) + b_ref[...]
  out_ref[...] = (
      jnp.dot(h, wc_ref[...], preferred_element_type=jnp.float32) + bc_ref[...]
      )


def _row_spec():
  return pl.BlockSpec((RB, D), lambda i: (i, 0))


def _full_spec(shape):
  return pl.BlockSpec(shape, lambda i: tuple(0 for _ in shape))


def kernel(x, edge_index, W_embed, b_embed, W1, b1, W2, b2, W_cls, b_cls):
  src = edge_index[0]
  dst = edge_index[1]
  be = b_embed.reshape(1, D)
  b1r = b1.reshape(1, D)
  b2r = b2.reshape(1, D)
  ncls = W_cls.shape[1]
  bcr = b_cls.reshape(1, ncls)

  degp = _deg_sc(dst)

  part_spec = pl.BlockSpec((NC, RB, D), lambda i: (0, i, 0))

  y1, dv = pl.pallas_call(
      _tc1_body,
      grid=(GRID,),
      in_specs=[
          _row_spec(),
          _full_spec((D, D)),
          _full_spec((1, D)),
          _full_spec((D, D)),
          pl.BlockSpec((NC, RB, 16), lambda i: (0, i, 0)),
      ],
      out_specs=[_row_spec(), _row_spec()],
      out_shape=[
          jax.ShapeDtypeStruct((N_NODES, D), jnp.float32),
          jax.ShapeDtypeStruct((N_NODES, D), jnp.float32),
      ],
  )(x, W_embed, be, W1, degp)

  p = _agg_sc(y1, src, dst)

  y2 = pl.pallas_call(
      _tc2_body,
      grid=(GRID,),
      in_specs=[
          part_spec, _row_spec(), _row_spec(),
          _full_spec((1, D)), _full_spec((D, D)),
      ],
      out_specs=_row_spec(),
      out_shape=jax.ShapeDtypeStruct((N_NODES, D), jnp.float32),
  )(p, y1, dv, b1r, W2)

  q = _agg_sc(y2, src, dst)

  logits = pl.pallas_call(
      _tc3_body,
      grid=(GRID,),
      in_specs=[
          part_spec, _row_spec(), _row_spec(),
          _full_spec((1, D)), _full_spec((D, ncls)), _full_spec((1, ncls)),
      ],
      out_specs=pl.BlockSpec((RB, ncls), lambda i: (i, 0)),
      out_shape=jax.ShapeDtypeStruct((N_NODES, ncls), jnp.float32),
  )(q, y2, dv, b2r, W_cls, bcr)

  return logits
